# baseline (speedup 1.0000x reference)
import jax
import jax.numpy as jnp
from jax.experimental import pallas as pl
from jax.experimental.pallas import tpu as pltpu

D_MODEL = 768
N_EXPERTS = 64
EP = 128
TOP_K = 8
BLOCK = 2048


def _router_body(x_ref, wt_ref, bias_ref, w_out_ref, idx_out_ref, parts_ref):
    x = x_ref[...]
    wt = wt_ref[...]
    logits = jnp.dot(x, wt, preferred_element_type=jnp.float32)
    logits = logits + bias_ref[...]

    lane_f = jax.lax.broadcasted_iota(jnp.int32, (BLOCK, EP), 1).astype(
        jnp.float32)
    cur = logits
    onehot_acc = jnp.zeros((BLOCK, EP), jnp.float32)
    m_cols = []
    idx_cols = []
    for k in range(TOP_K):
        m = jnp.max(cur, axis=1, keepdims=True)
        idx_f = jnp.min(jnp.where(cur == m, lane_f, jnp.float32(EP)),
                        axis=1, keepdims=True)
        onehot = (lane_f == idx_f)
        onehot_acc = onehot_acc + jnp.where(onehot, 1.0, 0.0)
        m_cols.append(m)
        idx_cols.append(idx_f)
        cur = jnp.where(onehot, -jnp.inf, cur)

    vals = jnp.concatenate(m_cols, axis=1)
    e = jnp.exp(vals - vals[:, :1])
    w_out_ref[...] = e / jnp.sum(e, axis=1, keepdims=True)
    idx_out_ref[...] = jnp.concatenate(idx_cols, axis=1).astype(jnp.int32)
    parts_ref[...] = jnp.sum(onehot_acc, axis=0, keepdims=True)[None]


def _stats_body(parts_ref, counts_ref, stats_ref):
    c = jnp.sum(parts_ref[...], axis=0)
    counts_ref[...] = c
    l0 = jax.lax.broadcasted_iota(jnp.int32, (1, EP), 1)
    valid = l0 < N_EXPERTS
    csum = jnp.sum(jnp.where(valid, c, 0.0))
    mean = csum / N_EXPERTS
    var = jnp.sum(jnp.where(valid, (c - mean) ** 2, 0.0)) / (N_EXPERTS - 1)
    lb = jnp.sqrt(var) / (mean + 1e-6)
    cmax = jnp.max(jnp.where(valid, c, -jnp.inf))
    cmin = jnp.min(jnp.where(valid, c, jnp.inf))
    stats_ref[...] = (jnp.where(l0 == 0, lb, 0.0)
                      + jnp.where(l0 == 1, cmax, 0.0)
                      + jnp.where(l0 == 2, cmin, 0.0))


def kernel(x, W, expert_bias):
    b, s, d = x.shape
    nt = b * s
    nb = nt // BLOCK
    x_flat = x.reshape(nt, d)
    wt = jnp.zeros((d, EP), jnp.float32).at[:, :N_EXPERTS].set(W.T)
    bias = jnp.full((1, EP), -jnp.inf, jnp.float32)
    bias = bias.at[0, :N_EXPERTS].set(expert_bias)

    w_out, idx_out, parts = pl.pallas_call(
        _router_body,
        grid=(nb,),
        in_specs=[
            pl.BlockSpec((BLOCK, d), lambda i: (i, 0)),
            pl.BlockSpec((d, EP), lambda i: (0, 0)),
            pl.BlockSpec((1, EP), lambda i: (0, 0)),
        ],
        out_specs=[
            pl.BlockSpec((BLOCK, TOP_K), lambda i: (i, 0)),
            pl.BlockSpec((BLOCK, TOP_K), lambda i: (i, 0)),
            pl.BlockSpec((1, 1, EP), lambda i: (i, 0, 0)),
        ],
        out_shape=[
            jax.ShapeDtypeStruct((nt, TOP_K), jnp.float32),
            jax.ShapeDtypeStruct((nt, TOP_K), jnp.int32),
            jax.ShapeDtypeStruct((nb, 1, EP), jnp.float32),
        ],
        compiler_params=pltpu.CompilerParams(
            dimension_semantics=("parallel",),
        ),
    )(x_flat, wt, bias)

    counts, stats = pl.pallas_call(
        _stats_body,
        out_shape=[
            jax.ShapeDtypeStruct((1, EP), jnp.float32),
            jax.ShapeDtypeStruct((1, EP), jnp.float32),
        ],
    )(parts)

    routing_weights = w_out.reshape(b, s, TOP_K)
    expert_indices = idx_out.reshape(b, s, TOP_K)
    expert_counts = counts[0, :N_EXPERTS]
    load_balance = stats[0, 0]
    cmax = stats[0, 1]
    cmin = stats[0, 2]
    expected_load = jnp.asarray(nt * TOP_K / N_EXPERTS, dtype=jnp.float32)
    return (routing_weights, expert_indices, expert_counts, load_balance,
            cmax, cmin, expected_load)


# speedup vs baseline: 1.0234x; 1.0234x over previous
import jax
import jax.numpy as jnp
from jax.experimental import pallas as pl
from jax.experimental.pallas import tpu as pltpu

D_MODEL = 768
N_EXPERTS = 64
EP = 128
TOP_K = 8
BLOCK = 4096


def _router_body(x_ref, wt_ref, bias_ref, w_out_ref, idx_out_ref,
                 counts_ref, stats_ref):
    i = pl.program_id(0)
    nsteps = pl.num_programs(0)

    x = x_ref[...]
    wt = wt_ref[...]
    logits = jnp.dot(x, wt, preferred_element_type=jnp.float32)
    logits = logits + bias_ref[...]

    lane_f = jax.lax.broadcasted_iota(jnp.int32, (BLOCK, EP), 1).astype(
        jnp.float32)
    cur = logits
    onehot_acc = jnp.zeros((BLOCK, EP), jnp.float32)
    m_cols = []
    idx_cols = []
    for k in range(TOP_K):
        m = jnp.max(cur, axis=1, keepdims=True)
        idx_f = jnp.min(jnp.where(cur == m, lane_f, jnp.float32(EP)),
                        axis=1, keepdims=True)
        onehot = (lane_f == idx_f)
        onehot_acc = onehot_acc + jnp.where(onehot, 1.0, 0.0)
        m_cols.append(m)
        idx_cols.append(idx_f)
        cur = jnp.where(onehot, -jnp.inf, cur)

    vals = jnp.concatenate(m_cols, axis=1)
    e = jnp.exp(vals - vals[:, :1])
    w_out_ref[...] = e / jnp.sum(e, axis=1, keepdims=True)
    idx_out_ref[...] = jnp.concatenate(idx_cols, axis=1).astype(jnp.int32)

    block_counts = jnp.sum(onehot_acc, axis=0, keepdims=True)

    @pl.when(i == 0)
    def _init():
        counts_ref[...] = block_counts

    @pl.when(i != 0)
    def _acc():
        counts_ref[...] = counts_ref[...] + block_counts

    @pl.when(i == nsteps - 1)
    def _stats():
        c = counts_ref[...]
        l0 = jax.lax.broadcasted_iota(jnp.int32, (1, EP), 1)
        valid = l0 < N_EXPERTS
        csum = jnp.sum(jnp.where(valid, c, 0.0))
        mean = csum / N_EXPERTS
        var = jnp.sum(jnp.where(valid, (c - mean) ** 2, 0.0)) / (N_EXPERTS - 1)
        lb = jnp.sqrt(var) / (mean + 1e-6)
        cmax = jnp.max(jnp.where(valid, c, -jnp.inf))
        cmin = jnp.min(jnp.where(valid, c, jnp.inf))
        stats_ref[...] = (jnp.where(l0 == 0, lb, 0.0)
                          + jnp.where(l0 == 1, cmax, 0.0)
                          + jnp.where(l0 == 2, cmin, 0.0))


def kernel(x, W, expert_bias):
    b, s, d = x.shape
    nt = b * s
    x_flat = x.reshape(nt, d)
    wt = jnp.zeros((d, EP), jnp.float32).at[:, :N_EXPERTS].set(W.T)
    bias = jnp.full((1, EP), -jnp.inf, jnp.float32)
    bias = bias.at[0, :N_EXPERTS].set(expert_bias)

    grid = (nt // BLOCK,)
    w_out, idx_out, counts, stats = pl.pallas_call(
        _router_body,
        grid=grid,
        in_specs=[
            pl.BlockSpec((BLOCK, d), lambda i: (i, 0)),
            pl.BlockSpec((d, EP), lambda i: (0, 0)),
            pl.BlockSpec((1, EP), lambda i: (0, 0)),
        ],
        out_specs=[
            pl.BlockSpec((BLOCK, TOP_K), lambda i: (i, 0)),
            pl.BlockSpec((BLOCK, TOP_K), lambda i: (i, 0)),
            pl.BlockSpec((1, EP), lambda i: (0, 0)),
            pl.BlockSpec((1, EP), lambda i: (0, 0)),
        ],
        out_shape=[
            jax.ShapeDtypeStruct((nt, TOP_K), jnp.float32),
            jax.ShapeDtypeStruct((nt, TOP_K), jnp.int32),
            jax.ShapeDtypeStruct((1, EP), jnp.float32),
            jax.ShapeDtypeStruct((1, EP), jnp.float32),
        ],
        compiler_params=pltpu.CompilerParams(
            dimension_semantics=("arbitrary",),
        ),
    )(x_flat, wt, bias)

    routing_weights = w_out.reshape(b, s, TOP_K)
    expert_indices = idx_out.reshape(b, s, TOP_K)
    expert_counts = counts[0, :N_EXPERTS]
    load_balance = stats[0, 0]
    cmax = stats[0, 1]
    cmin = stats[0, 2]
    expected_load = jnp.asarray(nt * TOP_K / N_EXPERTS, dtype=jnp.float32)
    return (routing_weights, expert_indices, expert_counts, load_balance,
            cmax, cmin, expected_load)


# speedup vs baseline: 1.0265x; 1.0030x over previous
import jax
import jax.numpy as jnp
from jax.experimental import pallas as pl
from jax.experimental.pallas import tpu as pltpu

D_MODEL = 768
N_EXPERTS = 64
EP = 128
TOP_K = 8
BLOCK = 4096
HALF = BLOCK // 2


def _router_body(x1_ref, x2_ref, wt_ref, bias_ref, w_out_ref, idx_out_ref,
                 counts_ref, stats_ref):
    i = pl.program_id(0)
    nsteps = pl.num_programs(0)

    wt = wt_ref[...]
    l1 = jnp.dot(x1_ref[...], wt, preferred_element_type=jnp.float32)
    l2 = jnp.dot(x2_ref[...], wt, preferred_element_type=jnp.float32)
    logits = jnp.concatenate([l1, l2], axis=0)
    logits = logits + bias_ref[...]

    lane_f = jax.lax.broadcasted_iota(jnp.int32, (BLOCK, EP), 1).astype(
        jnp.float32)
    cur = logits
    onehot_acc = jnp.zeros((BLOCK, EP), jnp.float32)
    m_cols = []
    idx_cols = []
    for k in range(TOP_K):
        m = jnp.max(cur, axis=1, keepdims=True)
        idx_f = jnp.min(jnp.where(cur == m, lane_f, jnp.float32(EP)),
                        axis=1, keepdims=True)
        onehot = (lane_f == idx_f)
        onehot_acc = onehot_acc + jnp.where(onehot, 1.0, 0.0)
        m_cols.append(m)
        idx_cols.append(idx_f)
        cur = jnp.where(onehot, -jnp.inf, cur)

    vals = jnp.concatenate(m_cols, axis=1)
    e = jnp.exp(vals - vals[:, :1])
    w_out_ref[...] = e / jnp.sum(e, axis=1, keepdims=True)
    idx_out_ref[...] = jnp.concatenate(idx_cols, axis=1).astype(jnp.int32)

    block_counts = jnp.sum(onehot_acc, axis=0, keepdims=True)

    @pl.when(i == 0)
    def _init():
        counts_ref[...] = block_counts

    @pl.when(i != 0)
    def _acc():
        counts_ref[...] = counts_ref[...] + block_counts

    @pl.when(i == nsteps - 1)
    def _stats():
        c = counts_ref[...]
        l0 = jax.lax.broadcasted_iota(jnp.int32, (1, EP), 1)
        valid = l0 < N_EXPERTS
        csum = jnp.sum(jnp.where(valid, c, 0.0))
        mean = csum / N_EXPERTS
        var = jnp.sum(jnp.where(valid, (c - mean) ** 2, 0.0)) / (N_EXPERTS - 1)
        lb = jnp.sqrt(var) / (mean + 1e-6)
        cmax = jnp.max(jnp.where(valid, c, -jnp.inf))
        cmin = jnp.min(jnp.where(valid, c, jnp.inf))
        stats_ref[...] = (jnp.where(l0 == 0, lb, 0.0)
                          + jnp.where(l0 == 1, cmax, 0.0)
                          + jnp.where(l0 == 2, cmin, 0.0))


def kernel(x, W, expert_bias):
    b, s, d = x.shape
    nt = b * s
    x_flat = x.reshape(nt, d)
    wt = jnp.zeros((d, EP), jnp.float32).at[:, :N_EXPERTS].set(W.T)
    bias = jnp.full((1, EP), -jnp.inf, jnp.float32)
    bias = bias.at[0, :N_EXPERTS].set(expert_bias)

    grid = (nt // BLOCK,)
    w_out, idx_out, counts, stats = pl.pallas_call(
        _router_body,
        grid=grid,
        in_specs=[
            pl.BlockSpec((HALF, d), lambda i: (2 * i, 0)),
            pl.BlockSpec((HALF, d), lambda i: (2 * i + 1, 0)),
            pl.BlockSpec((d, EP), lambda i: (0, 0)),
            pl.BlockSpec((1, EP), lambda i: (0, 0)),
        ],
        out_specs=[
            pl.BlockSpec((BLOCK, TOP_K), lambda i: (i, 0)),
            pl.BlockSpec((BLOCK, TOP_K), lambda i: (i, 0)),
            pl.BlockSpec((1, EP), lambda i: (0, 0)),
            pl.BlockSpec((1, EP), lambda i: (0, 0)),
        ],
        out_shape=[
            jax.ShapeDtypeStruct((nt, TOP_K), jnp.float32),
            jax.ShapeDtypeStruct((nt, TOP_K), jnp.int32),
            jax.ShapeDtypeStruct((1, EP), jnp.float32),
            jax.ShapeDtypeStruct((1, EP), jnp.float32),
        ],
        compiler_params=pltpu.CompilerParams(
            dimension_semantics=("arbitrary",),
        ),
    )(x_flat, x_flat, wt, bias)

    routing_weights = w_out.reshape(b, s, TOP_K)
    expert_indices = idx_out.reshape(b, s, TOP_K)
    expert_counts = counts[0, :N_EXPERTS]
    load_balance = stats[0, 0]
    cmax = stats[0, 1]
    cmin = stats[0, 2]
    expected_load = jnp.asarray(nt * TOP_K / N_EXPERTS, dtype=jnp.float32)
    return (routing_weights, expert_indices, expert_counts, load_balance,
            cmax, cmin, expected_load)
